# Initial kernel scaffold; baseline (speedup 1.0000x reference)
import jax
import jax.numpy as jnp
from jax.experimental import pallas as pl


def kernel(logits, targets, lengths):
    raise NotImplementedError("write your pallas kernel here")


# speedup vs baseline: 9548.9934x; 9548.9934x over previous
import jax
import jax.numpy as jnp
from jax.experimental import pallas as pl
from jax.experimental.pallas import tpu as pltpu

_TILE = 128
_NG = 8


def _pairwise_body(len_ref, x_ref, t_ref, out_ref, s_ref, ls_ref):
    x = x_ref[:, :]
    ls = -(jnp.maximum(-x, 0.0) + jnp.log1p(jnp.exp(-jnp.abs(x))))
    ls_ref[:, :] = ls
    s_ref[:, :] = jnp.exp(ls)

    iota_i = jax.lax.broadcasted_iota(jnp.int32, (_TILE, _TILE), 0)
    iota_j = jax.lax.broadcasted_iota(jnp.int32, (_TILE, _TILE), 1)
    ones_row = jnp.ones((1, _TILE), jnp.float32)

    def outer(v):
        return jax.lax.dot_general(
            v, ones_row, (((0,), (0,)), ((), ())),
            preferred_element_type=jnp.float32)

    acc = jnp.zeros((1, _TILE), jnp.float32)
    cnt = jnp.zeros((1, _TILE), jnp.int32)
    off = jnp.int32(0)
    for g in range(_NG):
        seg_len = len_ref[g]
        end = off + seg_len
        lo = off // _TILE
        hi = (end + _TILE - 1) // _TILE
        off_g = off

        def ti_body(ri, carry, off=off_g, end=end, lo=lo):
            si_row = s_ref[pl.ds(ri, 1), :]
            lsi_row = ls_ref[pl.ds(ri, 1), :]
            ti_row = t_ref[pl.ds(ri, 1), :]
            s_i = outer(si_row)
            ls_i = outer(lsi_row)
            t_i = outer(ti_row)
            gi = iota_i + ri * _TILE
            valid_i = (gi >= off) & (gi < end)

            def tj_body(rj, carry2):
                acc2, cnt2 = carry2
                sj_row = s_ref[pl.ds(rj, 1), :]
                lsj_row = ls_ref[pl.ds(rj, 1), :]
                tj_row = t_ref[pl.ds(rj, 1), :]
                gj = iota_j + rj * _TILE
                inb = valid_i & (gj >= off) & (gj < end)
                p = jnp.log(s_i + sj_row)
                m1 = inb & (t_i > tj_row)
                m2 = inb & (tj_row > t_i) & (rj != ri)
                loss = (jnp.where(m1, p - ls_i, 0.0)
                        + jnp.where(m2, p - lsj_row, 0.0))
                acc2 = acc2 + jnp.sum(loss, axis=0, keepdims=True)
                cnt2 = cnt2 + jnp.sum(
                    m1.astype(jnp.int32) + m2.astype(jnp.int32),
                    axis=0, keepdims=True)
                return acc2, cnt2

            return jax.lax.fori_loop(lo, ri + 1, tj_body, carry)

        acc, cnt = jax.lax.fori_loop(lo, hi, ti_body, (acc, cnt))
        off = end

    total = jnp.sum(acc)
    count = jnp.sum(cnt)
    out_ref[0, 0] = jnp.where(
        count > 0, total / count.astype(jnp.float32), 0.0)


def kernel(logits, targets, lengths):
    x2d = logits.reshape(_TILE, _TILE)
    t2d = targets.reshape(_TILE, _TILE)
    out = pl.pallas_call(
        _pairwise_body,
        out_shape=jax.ShapeDtypeStruct((1, 1), jnp.float32),
        in_specs=[
            pl.BlockSpec(memory_space=pltpu.SMEM),
            pl.BlockSpec(memory_space=pltpu.VMEM),
            pl.BlockSpec(memory_space=pltpu.VMEM),
        ],
        out_specs=pl.BlockSpec(memory_space=pltpu.SMEM),
        scratch_shapes=[
            pltpu.VMEM((_TILE, _TILE), jnp.float32),
            pltpu.VMEM((_TILE, _TILE), jnp.float32),
        ],
    )(lengths, x2d, t2d)
    return out[0, 0]


# speedup vs baseline: 27204.7256x; 2.8490x over previous
import jax
import jax.numpy as jnp
from jax.experimental import pallas as pl
from jax.experimental.pallas import tpu as pltpu

_TILE = 128
_NG = 8


def _pairwise_body(len_ref, x_ref, t_ref, out_ref, s_ref, ls_ref):
    x = x_ref[:, :]
    ls = -(jnp.maximum(-x, 0.0) + jnp.log1p(jnp.exp(-jnp.abs(x))))
    ls_ref[:, :] = ls
    s_ref[:, :] = jnp.exp(ls)

    iota_i = jax.lax.broadcasted_iota(jnp.int32, (_TILE, _TILE), 0)
    iota_j1 = jax.lax.broadcasted_iota(jnp.int32, (1, _TILE), 1)
    ones_row = jnp.ones((1, _TILE), jnp.float32)
    onef = jnp.float32(1.0)
    zerof = jnp.float32(0.0)

    def outer(v):
        return jax.lax.dot_general(
            v, ones_row, (((0,), (0,)), ((), ())),
            preferred_element_type=jnp.float32)

    def fold(v):
        return jnp.sum(v.reshape(16, 8, _TILE), axis=0)

    acc = jnp.zeros((8, _TILE), jnp.float32)
    cnt = jnp.zeros((8, _TILE), jnp.float32)
    off = jnp.int32(0)
    for g in range(_NG):
        end = off + len_ref[g]
        lo = off // _TILE
        hi = (end + _TILE - 1) // _TILE
        off_g = off

        def ti_body(ri, carry, off=off_g, end=end, lo=lo):
            acc1, cnt1 = carry
            si_row = s_ref[pl.ds(ri, 1), :]
            lsi_row = ls_ref[pl.ds(ri, 1), :]
            ti_row = t_ref[pl.ds(ri, 1), :]
            s_i = outer(si_row)
            ls_i = outer(lsi_row)
            t_i = outer(ti_row)
            gi = iota_i + ri * _TILE
            valid_i = (gi >= off) & (gi < end)

            def tile(rj, jpred, acc2, cnt2):
                sj_row = s_ref[pl.ds(rj, 1), :]
                lsj_row = ls_ref[pl.ds(rj, 1), :]
                tj_row = t_ref[pl.ds(rj, 1), :]
                gj = iota_j1 + rj * _TILE
                jm = (gj >= off) & (gj < end) & jpred
                inb = valid_i & jm
                p = jnp.log(s_i + sj_row)
                m1f = jnp.where(inb & (t_i > tj_row), onef, zerof)
                m2f = jnp.where(inb & (tj_row > t_i), onef, zerof)
                cf = m1f + m2f
                contrib = cf * p - m1f * ls_i - m2f * lsj_row
                return acc2 + fold(contrib), cnt2 + fold(cf)

            def diag_tile(rj, acc2, cnt2):
                sj_row = s_ref[pl.ds(rj, 1), :]
                tj_row = t_ref[pl.ds(rj, 1), :]
                gj = iota_j1 + rj * _TILE
                jm = (gj >= off) & (gj < end)
                inb = valid_i & jm
                p = jnp.log(s_i + sj_row)
                m1f = jnp.where(inb & (t_i > tj_row), onef, zerof)
                return acc2 + fold(m1f * (p - ls_i)), cnt2 + fold(m1f)

            acc1, cnt1 = diag_tile(ri, acc1, cnt1)

            def tj_body(k, carry2):
                acc2, cnt2 = carry2
                rj = lo + 2 * k
                acc2, cnt2 = tile(rj, True, acc2, cnt2)
                acc2, cnt2 = tile(rj + 1, rj + 1 < ri, acc2, cnt2)
                return acc2, cnt2

            npairs = ri - lo
            return jax.lax.fori_loop(
                0, (npairs + 1) // 2, tj_body, (acc1, cnt1))

        acc, cnt = jax.lax.fori_loop(lo, hi, ti_body, (acc, cnt))
        off = end

    total = jnp.sum(acc)
    count = jnp.sum(cnt.astype(jnp.int32))
    out_ref[0, 0] = jnp.where(
        count > 0, total / count.astype(jnp.float32), 0.0)


def kernel(logits, targets, lengths):
    x2d = logits.reshape(_TILE, _TILE)
    t2d = targets.reshape(_TILE, _TILE)
    out = pl.pallas_call(
        _pairwise_body,
        out_shape=jax.ShapeDtypeStruct((1, 1), jnp.float32),
        in_specs=[
            pl.BlockSpec(memory_space=pltpu.SMEM),
            pl.BlockSpec(memory_space=pltpu.VMEM),
            pl.BlockSpec(memory_space=pltpu.VMEM),
        ],
        out_specs=pl.BlockSpec(memory_space=pltpu.SMEM),
        scratch_shapes=[
            pltpu.VMEM((_TILE, _TILE), jnp.float32),
            pltpu.VMEM((_TILE, _TILE), jnp.float32),
        ],
    )(lengths, x2d, t2d)
    return out[0, 0]
